# baseline (speedup 1.0000x reference)
import functools

import jax
import jax.numpy as jnp
from jax import lax
from jax.experimental import pallas as pl
from jax.experimental.pallas import tpu as pltpu
from jax.experimental.pallas import tpu_sc as plsc

BQ, VQ, LQ, CQ, NQ = 8, 4, 256, 256, 5000
NMAPS = BQ * VQ
NP = 5024
G = 16
NCHUNKS = NP // G
H = 16
SCALE = H * 1.0 / 224.0

_mesh = plsc.VectorSubcoreMesh(
    core_axis_name="c", subcore_axis_name="s", num_cores=2, num_subcores=16
)


def _body(feats_hbm, boxes_hbm, out_hbm, map_v, box_v, out_v, sem0, sem1):
    wid = lax.axis_index("s") * 2 + lax.axis_index("c")
    pltpu.sync_copy(feats_hbm.at[wid], map_v)
    pltpu.sync_copy(boxes_hbm.at[wid], box_v)

    def chunk(k, buf, sem):
        g16 = pl.ds(k * G, G)
        bx1 = box_v[0, g16]
        by1 = box_v[1, g16]
        bx2 = box_v[2, g16]
        by2 = box_v[3, g16]
        cx = (bx1 + bx2) * 0.5
        cy = (by1 + by2) * 0.5
        nsx = (bx2 - bx1) * 1.1
        nsy = (by2 - by1) * 1.1
        lox = jnp.maximum(cx - nsx * 0.5, 0.0)
        loy = jnp.maximum(cy - nsy * 0.5, 0.0)
        hix = jnp.minimum(cx + nsx * 0.5, 224.0)
        hiy = jnp.minimum(cy + nsy * 0.5, 224.0)
        hix = jnp.maximum(hix, lox + 1e-6)
        hiy = jnp.maximum(hiy, loy + 1e-6)
        lox = jnp.minimum(lox, 224.0)
        loy = jnp.minimum(loy, 224.0)
        hix = jnp.minimum(hix, 224.0)
        hiy = jnp.minimum(hiy, 224.0)
        x1s = lox * SCALE
        y1s = loy * SCALE
        x2s = hix * SCALE
        y2s = hiy * SCALE
        roi_w = jnp.maximum(x2s - x1s, 1.0)
        roi_h = jnp.maximum(y2s - y1s, 1.0)
        sx = x1s + 0.5 * roi_w
        sy = y1s + 0.5 * roi_h
        sx = jnp.minimum(jnp.maximum(sx, 0.0), H - 1.0)
        sy = jnp.minimum(jnp.maximum(sy, 0.0), H - 1.0)
        x0 = jnp.minimum(sx.astype(jnp.int32), H - 2)
        y0 = jnp.minimum(sy.astype(jnp.int32), H - 2)
        lx = sx - x0.astype(jnp.float32)
        ly = sy - y0.astype(jnp.float32)
        hx = 1.0 - lx
        hy = 1.0 - ly
        o00v = (y0 * H + x0) * CQ
        w00v = hy * hx
        w01v = hy * lx
        w10v = ly * hx
        w11v = ly * lx

        for i in range(G):
            o00 = o00v[i]
            w00 = jnp.full((16,), w00v[i], jnp.float32)
            w01 = jnp.full((16,), w01v[i], jnp.float32)
            w10 = jnp.full((16,), w10v[i], jnp.float32)
            w11 = jnp.full((16,), w11v[i], jnp.float32)
            for j in range(CQ // 16):
                a = map_v[pl.ds(o00 + j * 16, 16)]
                b = map_v[pl.ds(o00 + CQ + j * 16, 16)]
                c = map_v[pl.ds(o00 + H * CQ + j * 16, 16)]
                d = map_v[pl.ds(o00 + (H + 1) * CQ + j * 16, 16)]
                out_v[buf, i, pl.ds(j * 16, 16)] = (w00 * a + w01 * b) + (
                    w10 * c + w11 * d
                )

        pltpu.async_copy(out_v.at[buf], out_hbm.at[wid, pl.ds(k * G, G)], sem)

    def drain(buf, sem):
        pltpu.make_async_copy(
            out_v.at[buf], out_hbm.at[wid, pl.ds(0, G)], sem
        ).wait()

    def pair(k2, carry):
        pl.when(k2 > 0)(lambda: drain(0, sem0))
        chunk(k2 * 2, 0, sem0)
        pl.when(k2 > 0)(lambda: drain(1, sem1))
        chunk(k2 * 2 + 1, 1, sem1)
        return carry

    lax.fori_loop(0, NCHUNKS // 2, pair, 0)
    drain(0, sem0)
    drain(1, sem1)


_sc_call = pl.kernel(
    _body,
    out_type=jax.ShapeDtypeStruct((NMAPS, NP, CQ), jnp.float32),
    mesh=_mesh,
    scratch_types=[
        pltpu.VMEM((LQ * CQ,), jnp.float32),
        pltpu.VMEM((4, NP), jnp.float32),
        pltpu.VMEM((2, G, CQ), jnp.float32),
        pltpu.SemaphoreType.DMA,
        pltpu.SemaphoreType.DMA,
    ],
)


@jax.jit
def kernel(img_feats, bboxes):
    feats = img_feats.reshape(NMAPS, LQ * CQ)
    boxes = bboxes.reshape(NMAPS, NQ, 4)
    boxes = jnp.concatenate(
        [boxes, jnp.zeros((NMAPS, NP - NQ, 4), jnp.float32)], axis=1
    )
    boxes_t = boxes.transpose(0, 2, 1)
    out = _sc_call(feats, boxes_t)
    return out[:, :NQ].reshape(BQ, VQ, NQ, CQ)


# speedup vs baseline: 2.9194x; 2.9194x over previous
import functools

import jax
import jax.numpy as jnp
from jax import lax
from jax.experimental import pallas as pl
from jax.experimental.pallas import tpu as pltpu
from jax.experimental.pallas import tpu_sc as plsc

BQ, VQ, LQ, CQ, NQ = 8, 4, 256, 256, 5000
NMAPS = BQ * VQ
NP = 5024
G = 16
NCHUNKS = NP // G
H = 16
SCALE = H * 1.0 / 224.0

_mesh = plsc.VectorSubcoreMesh(
    core_axis_name="c", subcore_axis_name="s", num_cores=2, num_subcores=16
)


def _body(feats_hbm, boxes_hbm, out_hbm, map_v, box_v, out_v):
    wid = lax.axis_index("s") * 2 + lax.axis_index("c")
    pltpu.sync_copy(feats_hbm.at[wid], map_v)
    pltpu.sync_copy(boxes_hbm.at[wid], box_v)

    def chunk(k, carry):
        g16 = pl.ds(k * G, G)
        bx1 = box_v[0, g16]
        by1 = box_v[1, g16]
        bx2 = box_v[2, g16]
        by2 = box_v[3, g16]
        cx = (bx1 + bx2) * 0.5
        cy = (by1 + by2) * 0.5
        nsx = (bx2 - bx1) * 1.1
        nsy = (by2 - by1) * 1.1
        lox = jnp.maximum(cx - nsx * 0.5, 0.0)
        loy = jnp.maximum(cy - nsy * 0.5, 0.0)
        hix = jnp.minimum(cx + nsx * 0.5, 224.0)
        hiy = jnp.minimum(cy + nsy * 0.5, 224.0)
        hix = jnp.maximum(hix, lox + 1e-6)
        hiy = jnp.maximum(hiy, loy + 1e-6)
        lox = jnp.minimum(lox, 224.0)
        loy = jnp.minimum(loy, 224.0)
        hix = jnp.minimum(hix, 224.0)
        hiy = jnp.minimum(hiy, 224.0)
        x1s = lox * SCALE
        y1s = loy * SCALE
        x2s = hix * SCALE
        y2s = hiy * SCALE
        roi_w = jnp.maximum(x2s - x1s, 1.0)
        roi_h = jnp.maximum(y2s - y1s, 1.0)
        sx = x1s + 0.5 * roi_w
        sy = y1s + 0.5 * roi_h
        sx = jnp.minimum(jnp.maximum(sx, 0.0), H - 1.0)
        sy = jnp.minimum(jnp.maximum(sy, 0.0), H - 1.0)
        x0 = jnp.minimum(sx.astype(jnp.int32), H - 2)
        y0 = jnp.minimum(sy.astype(jnp.int32), H - 2)
        lx = sx - x0.astype(jnp.float32)
        ly = sy - y0.astype(jnp.float32)
        hx = 1.0 - lx
        hy = 1.0 - ly
        o00v = (y0 * H + x0) * CQ
        w00v = hy * hx
        w01v = hy * lx
        w10v = ly * hx
        w11v = ly * lx

        for i in range(G):
            o00 = o00v[i]
            w00 = jnp.full((16,), w00v[i], jnp.float32)
            w01 = jnp.full((16,), w01v[i], jnp.float32)
            w10 = jnp.full((16,), w10v[i], jnp.float32)
            w11 = jnp.full((16,), w11v[i], jnp.float32)
            accs = []
            for j in range(CQ // 16):
                a = map_v[pl.ds(o00 + j * 16, 16)]
                b = map_v[pl.ds(o00 + CQ + j * 16, 16)]
                c = map_v[pl.ds(o00 + H * CQ + j * 16, 16)]
                d = map_v[pl.ds(o00 + (H + 1) * CQ + j * 16, 16)]
                accs.append((w00 * a + w01 * b) + (w10 * c + w11 * d))
            for j in range(CQ // 16):
                out_v[i, pl.ds(j * 16, 16)] = accs[j]

        pltpu.sync_copy(out_v, out_hbm.at[wid, pl.ds(k * G, G)])
        return 0

    lax.fori_loop(0, NCHUNKS, chunk, 0)


_sc_call = pl.kernel(
    _body,
    out_type=jax.ShapeDtypeStruct((NMAPS, NP, CQ), jnp.float32),
    mesh=_mesh,
    scratch_types=[
        pltpu.VMEM((LQ * CQ,), jnp.float32),
        pltpu.VMEM((4, NP), jnp.float32),
        pltpu.VMEM((G, CQ), jnp.float32),
    ],
)


@jax.jit
def kernel(img_feats, bboxes):
    feats = img_feats.reshape(NMAPS, LQ * CQ)
    boxes = bboxes.reshape(NMAPS, NQ, 4)
    boxes = jnp.concatenate(
        [boxes, jnp.zeros((NMAPS, NP - NQ, 4), jnp.float32)], axis=1
    )
    boxes_t = boxes.transpose(0, 2, 1)
    out = _sc_call(feats, boxes_t)
    return out[:, :NQ].reshape(BQ, VQ, NQ, CQ)


# speedup vs baseline: 3.7216x; 1.2748x over previous
import functools

import jax
import jax.numpy as jnp
import numpy as np
from jax import lax
from jax.experimental import pallas as pl
from jax.experimental.pallas import tpu as pltpu
from jax.experimental.pallas import tpu_sc as plsc

BQ, VQ, LQ, CQ, NQ = 8, 4, 256, 256, 5000
NMAPS = BQ * VQ
NP = 5024
G = 16
NCHUNKS = NP // G
H = 16
SCALE = H * 1.0 / 224.0

_mesh = plsc.VectorSubcoreMesh(
    core_axis_name="c", subcore_axis_name="s", num_cores=2, num_subcores=16
)


def _rnd16(u):
    return (u + jnp.uint32(0x7FFF) + ((u >> 16) & jnp.uint32(1))) >> 16


def _body(feats_hbm, boxes_hbm, out_hbm, map_v, box_v, out_v):
    wid = lax.axis_index("s") * 2 + lax.axis_index("c")
    pltpu.sync_copy(feats_hbm.at[wid], map_v)
    pltpu.sync_copy(boxes_hbm.at[wid], box_v)

    def chunk(k, carry):
        g16 = pl.ds(k * G, G)
        bx1 = box_v[0, g16]
        by1 = box_v[1, g16]
        bx2 = box_v[2, g16]
        by2 = box_v[3, g16]
        cx = (bx1 + bx2) * 0.5
        cy = (by1 + by2) * 0.5
        nsx = (bx2 - bx1) * 1.1
        nsy = (by2 - by1) * 1.1
        lox = jnp.maximum(cx - nsx * 0.5, 0.0)
        loy = jnp.maximum(cy - nsy * 0.5, 0.0)
        hix = jnp.minimum(cx + nsx * 0.5, 224.0)
        hiy = jnp.minimum(cy + nsy * 0.5, 224.0)
        hix = jnp.maximum(hix, lox + 1e-6)
        hiy = jnp.maximum(hiy, loy + 1e-6)
        lox = jnp.minimum(lox, 224.0)
        loy = jnp.minimum(loy, 224.0)
        hix = jnp.minimum(hix, 224.0)
        hiy = jnp.minimum(hiy, 224.0)
        x1s = lox * SCALE
        y1s = loy * SCALE
        x2s = hix * SCALE
        y2s = hiy * SCALE
        roi_w = jnp.maximum(x2s - x1s, 1.0)
        roi_h = jnp.maximum(y2s - y1s, 1.0)
        sx = x1s + 0.5 * roi_w
        sy = y1s + 0.5 * roi_h
        sx = jnp.minimum(jnp.maximum(sx, 0.0), H - 1.0)
        sy = jnp.minimum(jnp.maximum(sy, 0.0), H - 1.0)
        x0 = jnp.minimum(sx.astype(jnp.int32), H - 2)
        y0 = jnp.minimum(sy.astype(jnp.int32), H - 2)
        lx = sx - x0.astype(jnp.float32)
        ly = sy - y0.astype(jnp.float32)
        hx = 1.0 - lx
        hy = 1.0 - ly
        o00v = (y0 * H + x0) * (CQ // 2)
        w00r = _rnd16(plsc.bitcast(hy * hx, jnp.uint32))
        w01r = _rnd16(plsc.bitcast(hy * lx, jnp.uint32))
        w10r = _rnd16(plsc.bitcast(ly * hx, jnp.uint32))
        w11r = _rnd16(plsc.bitcast(ly * lx, jnp.uint32))
        w00r = w00r | (w00r << 16)
        w01r = w01r | (w01r << 16)
        w10r = w10r | (w10r << 16)
        w11r = w11r | (w11r << 16)

        for i in range(G):
            o00 = o00v[i]

            def bsplat(wr):
                return plsc.bitcast(
                    jnp.full((16,), wr[i], jnp.uint32), jnp.bfloat16
                )

            w00 = bsplat(w00r)
            w01 = bsplat(w01r)
            w10 = bsplat(w10r)
            w11 = bsplat(w11r)
            accs = []
            CW = CQ // 2
            for j in range(CQ // 32):
                a = map_v[pl.ds(o00 + j * 16, 16)]
                b = map_v[pl.ds(o00 + CW + j * 16, 16)]
                c = map_v[pl.ds(o00 + H * CW + j * 16, 16)]
                d = map_v[pl.ds(o00 + (H + 1) * CW + j * 16, 16)]
                ab = plsc.bitcast(a, jnp.bfloat16)
                bb = plsc.bitcast(b, jnp.bfloat16)
                cb = plsc.bitcast(c, jnp.bfloat16)
                db = plsc.bitcast(d, jnp.bfloat16)
                acc = (w00 * ab + w01 * bb) + (w10 * cb + w11 * db)
                u = plsc.bitcast(acc, jnp.uint32)
                lo = plsc.bitcast(u << 16, jnp.float32)
                hi = plsc.bitcast(u & jnp.uint32(0xFFFF0000), jnp.float32)
                accs.append((lo, hi))
            for j in range(CQ // 32):
                lo, hi = accs[j]
                out_v[i, pl.ds(j * 32, 16)] = lo
                out_v[i, pl.ds(j * 32 + 16, 16)] = hi

        pltpu.sync_copy(out_v, out_hbm.at[wid, pl.ds(k * G, G)])
        return 0

    lax.fori_loop(0, NCHUNKS, chunk, 0)


_sc_call = pl.kernel(
    _body,
    out_type=jax.ShapeDtypeStruct((NMAPS, NP, CQ), jnp.float32),
    mesh=_mesh,
    scratch_types=[
        pltpu.VMEM((LQ * CQ // 2,), jnp.uint32),
        pltpu.VMEM((4, NP), jnp.float32),
        pltpu.VMEM((G, CQ), jnp.float32),
    ],
    compiler_params=pltpu.CompilerParams(needs_layout_passes=False),
)


@jax.jit
def kernel(img_feats, bboxes):
    u = jax.lax.bitcast_convert_type(
        img_feats.reshape(NMAPS, LQ, CQ // 32, 32), jnp.uint32
    )
    r = (u + jnp.uint32(0x7FFF) + ((u >> 16) & jnp.uint32(1))) >> 16
    feats = (r[..., :16] | (r[..., 16:] << 16)).reshape(NMAPS, LQ * CQ // 2)
    boxes = bboxes.reshape(NMAPS, NQ, 4)
    boxes = jnp.concatenate(
        [boxes, jnp.zeros((NMAPS, NP - NQ, 4), jnp.float32)], axis=1
    )
    boxes_t = boxes.transpose(0, 2, 1)
    out = _sc_call(feats, boxes_t)
    return out[:, :NQ].reshape(BQ, VQ, NQ, CQ)
